# baseline (speedup 1.0000x reference)
import functools

import jax
import jax.numpy as jnp
from jax import lax
from jax.experimental import pallas as pl
from jax.experimental.pallas import tpu as pltpu
from jax.experimental.pallas import tpu_sc as plsc

B = 4096 * 200
D = 256
V = 4101
HD = D // 2

NC = 2
NS = 16
BPW = B // NS
C = 64
NCHUNK = BPW // C
NBUF = 8
NROUND = NCHUNK // NBUF - 1

TPT = 256
TPT_LAST = V - (NS - 1) * TPT

_mesh = plsc.VectorSubcoreMesh(core_axis_name="c", subcore_axis_name="s")


@functools.partial(
    pl.kernel,
    out_type=jax.ShapeDtypeStruct((B, D), jnp.float32),
    mesh=_mesh,
    scratch_types=[
        pltpu.VMEM((NBUF, C), jnp.int32),
        pltpu.VMEM((NBUF, C, HD), jnp.float32),
        pltpu.VMEM_SHARED((V, HD), jnp.float32),
        [pltpu.SemaphoreType.DMA] * NBUF,
        [pltpu.SemaphoreType.DMA] * NBUF,
    ],
)
def _gather_rows(idx_hbm, table_hbm, out_hbm, idx_v, rows_v, table_sp, sem_g, sem_w):
    cid = lax.axis_index("c")
    sid = lax.axis_index("s")
    base = sid * BPW
    col = cid * HD

    @pl.when(sid < NS - 1)
    def _():
        pltpu.sync_copy(
            table_hbm.at[pl.ds(sid * TPT, TPT), pl.ds(col, HD)],
            table_sp.at[pl.ds(sid * TPT, TPT)],
        )

    @pl.when(sid == NS - 1)
    def _():
        pltpu.sync_copy(
            table_hbm.at[pl.ds((NS - 1) * TPT, TPT_LAST), pl.ds(col, HD)],
            table_sp.at[pl.ds((NS - 1) * TPT, TPT_LAST)],
        )

    plsc.subcore_barrier()

    def start_gather(b):
        pltpu.async_copy(table_sp.at[idx_v.at[b]], rows_v.at[b], sem_g[b])

    def wait_gather(b):
        pltpu.make_async_copy(
            table_sp.at[idx_v.at[b]], rows_v.at[b], sem_g[b]
        ).wait()

    def start_write(b, g):
        pltpu.async_copy(
            rows_v.at[b],
            out_hbm.at[pl.ds(base + g * C, C), pl.ds(col, HD)],
            sem_w[b],
        )

    def wait_write(b):
        pltpu.make_async_copy(
            rows_v.at[b], out_hbm.at[pl.ds(base, C), pl.ds(col, HD)], sem_w[b]
        ).wait()

    def load_idx(b, g):
        pltpu.sync_copy(idx_hbm.at[pl.ds(base + g * C, C)], idx_v.at[b])

    for b in range(NBUF):
        load_idx(b, b)
        start_gather(b)

    def round_body(r, carry):
        for b in range(NBUF):
            g = r * NBUF + b
            wait_gather(b)
            start_write(b, g)
            load_idx(b, g + NBUF)
            wait_write(b)
            start_gather(b)
        return carry

    lax.fori_loop(0, NROUND, round_body, 0)

    for b in range(NBUF):
        g = NROUND * NBUF + b
        wait_gather(b)
        start_write(b, g)
        wait_write(b)


def kernel(action_indices, table):
    idx = action_indices.reshape(-1)
    out = _gather_rows(idx, table)
    return out.reshape(action_indices.shape + (table.shape[1],))


# speedup vs baseline: 1.2233x; 1.2233x over previous
import functools

import jax
import jax.numpy as jnp
from jax import lax
from jax.experimental import pallas as pl
from jax.experimental.pallas import tpu as pltpu
from jax.experimental.pallas import tpu_sc as plsc

B = 4096 * 200
D = 256
V = 4101
HD = D // 2

NC = 2
NS = 16
BPW = B // NS
C = 128
W = 2 * C
NSUP = BPW // W
NBUF = 2
NROUND = NSUP // NBUF - 1

TPT = 256
TPT_LAST = V - (NS - 1) * TPT

_mesh = plsc.VectorSubcoreMesh(core_axis_name="c", subcore_axis_name="s")


@functools.partial(
    pl.kernel,
    out_type=jax.ShapeDtypeStruct((B, D), jnp.float32),
    mesh=_mesh,
    scratch_types=[
        pltpu.VMEM((NBUF, W), jnp.int32),
        pltpu.VMEM((NBUF, W, HD), jnp.float32),
        pltpu.VMEM_SHARED((V, HD), jnp.float32),
        [pltpu.SemaphoreType.DMA] * NBUF,
        [pltpu.SemaphoreType.DMA] * NBUF,
    ],
)
def _gather_rows(idx_hbm, table_hbm, out_hbm, idx_v, rows_v, table_sp, sem_g, sem_w):
    cid = lax.axis_index("c")
    sid = lax.axis_index("s")
    base = sid * BPW
    col = cid * HD

    @pl.when(sid < NS - 1)
    def _():
        pltpu.sync_copy(
            table_hbm.at[pl.ds(sid * TPT, TPT), pl.ds(col, HD)],
            table_sp.at[pl.ds(sid * TPT, TPT)],
        )

    @pl.when(sid == NS - 1)
    def _():
        pltpu.sync_copy(
            table_hbm.at[pl.ds((NS - 1) * TPT, TPT_LAST), pl.ds(col, HD)],
            table_sp.at[pl.ds((NS - 1) * TPT, TPT_LAST)],
        )

    plsc.subcore_barrier()

    def start_gather(b):
        for h in range(2):
            pltpu.async_copy(
                table_sp.at[idx_v.at[b].at[pl.ds(h * C, C)]],
                rows_v.at[b].at[pl.ds(h * C, C)],
                sem_g[b],
            )

    def wait_gather(b):
        for h in range(2):
            pltpu.make_async_copy(
                table_sp.at[idx_v.at[b].at[pl.ds(h * C, C)]],
                rows_v.at[b].at[pl.ds(h * C, C)],
                sem_g[b],
            ).wait()

    def start_write(b, g):
        pltpu.async_copy(
            rows_v.at[b],
            out_hbm.at[pl.ds(base + g * W, W), pl.ds(col, HD)],
            sem_w[b],
        )

    def wait_write(b):
        pltpu.make_async_copy(
            rows_v.at[b], out_hbm.at[pl.ds(base, W), pl.ds(col, HD)], sem_w[b]
        ).wait()

    def load_idx(b, g):
        pltpu.sync_copy(idx_hbm.at[pl.ds(base + g * W, W)], idx_v.at[b])

    for b in range(NBUF):
        load_idx(b, b)
        start_gather(b)

    def round_body(r, carry):
        for b in range(NBUF):
            g = r * NBUF + b
            wait_gather(b)
            start_write(b, g)
            load_idx(b, g + NBUF)
            wait_write(b)
            start_gather(b)
        return carry

    lax.fori_loop(0, NROUND, round_body, 0)

    for b in range(NBUF):
        g = NROUND * NBUF + b
        wait_gather(b)
        start_write(b, g)
        wait_write(b)


def kernel(action_indices, table):
    idx = action_indices.reshape(-1)
    out = _gather_rows(idx, table)
    return out.reshape(action_indices.shape + (table.shape[1],))
